# baseline (speedup 1.0000x reference)
import functools

import jax
import jax.numpy as jnp
from jax import lax
from jax.experimental import pallas as pl
from jax.experimental.pallas import tpu as pltpu
from jax.experimental.pallas import tpu_sc as plsc

_NC = 2
_NS = 16
_NW = _NC * _NS
_L = 16


@functools.partial(jax.jit, static_argnums=(3, 4, 5))
def _embed_lookup(idx_flat, tok_table, pos_table, B, T, D):
    n_tok = B * T
    V = tok_table.shape[0]
    b_per_w = n_tok // _NW
    mesh = plsc.VectorSubcoreMesh(core_axis_name="c", subcore_axis_name="s")

    blk = V // 125

    def widen_body(tok_ref, ctab_ref):
        ctab_ref[:, 0:D] = tok_ref[...]

    widen = pl.pallas_call(
        widen_body,
        grid=(V // blk,),
        in_specs=[pl.BlockSpec((blk, D), lambda i: (i, 0))],
        out_specs=pl.BlockSpec((blk, 2 * D), lambda i: (i, 0)),
        out_shape=jax.ShapeDtypeStruct((V, 2 * D), jnp.float32),
    )

    @functools.partial(
        pl.kernel,
        out_type=jax.ShapeDtypeStruct((n_tok, D), jnp.float32),
        mesh=mesh,
        scratch_types=[
            pltpu.VMEM((b_per_w,), jnp.int32),
            pltpu.VMEM((b_per_w, 2 * D), jnp.float32),
            pltpu.VMEM((b_per_w, D), jnp.float32),
            pltpu.VMEM((b_per_w, D), jnp.float32),
            pltpu.SemaphoreType.DMA,
            pltpu.SemaphoreType.DMA,
        ],
    )
    def gather(idx_hbm, ctab_hbm, pos_hbm, out_hbm,
               idx_v, slots_v, out_v, pos_v, sem_g, sem_p):
        wid = lax.axis_index("s") * _NC + lax.axis_index("c")
        base = wid * b_per_w
        t0 = lax.rem(base, T)

        pltpu.sync_copy(idx_hbm.at[pl.ds(base, b_per_w)], idx_v)
        pos_cp = pltpu.async_copy(pos_hbm.at[pl.ds(t0, b_per_w)], pos_v, sem_p)
        gathers = []
        for k in range(b_per_w // 128):
            gathers.append(pltpu.async_copy(
                ctab_hbm.at[idx_v.at[pl.ds(k * 128, 128)]],
                slots_v.at[pl.ds(k * 128, 128)],
                sem_g,
            ))
        pos_cp.wait()
        for g in gathers:
            g.wait()

        def row_add(i, carry):
            for j in range(D // _L):
                s = pl.ds(j * _L, _L)
                out_v[i, s] = slots_v[i, s] + pos_v[i, s]
            return carry
        lax.fori_loop(0, b_per_w, row_add, 0)

        pltpu.sync_copy(out_v, out_hbm.at[pl.ds(base, b_per_w)])

    ctab = widen(tok_table)
    return gather(idx_flat, ctab, pos_table)


def kernel(idx, tok_table, pos_table):
    B, T = idx.shape
    V, D = tok_table.shape
    idx_flat = idx.reshape(-1).astype(jnp.int32)
    out = _embed_lookup(idx_flat, tok_table, pos_table, B, T, D)
    return out.reshape(B, T, D)


# speedup vs baseline: 1.0943x; 1.0943x over previous
import functools

import jax
import jax.numpy as jnp
from jax import lax
from jax.experimental import pallas as pl
from jax.experimental.pallas import tpu as pltpu
from jax.experimental.pallas import tpu_sc as plsc

_NC = 2
_NS = 16
_NW = _NC * _NS
_L = 16


@functools.partial(jax.jit, static_argnums=(3, 4, 5))
def _embed_lookup(idx_flat, tok_table, pos_table, B, T, D):
    n_tok = B * T
    V = tok_table.shape[0]
    b_per_w = n_tok // _NW
    mesh = plsc.VectorSubcoreMesh(core_axis_name="c", subcore_axis_name="s")

    @functools.partial(
        pl.kernel,
        out_type=jax.ShapeDtypeStruct((n_tok, D), jnp.float32),
        mesh=mesh,
        scratch_types=[
            pltpu.VMEM((b_per_w,), jnp.int32),
            pltpu.VMEM((b_per_w,), jnp.int32),
            pltpu.VMEM((b_per_w, 2 * D), jnp.float32),
            pltpu.VMEM((b_per_w, D), jnp.float32),
            pltpu.VMEM((b_per_w, D), jnp.float32),
            pltpu.SemaphoreType.DMA,
            pltpu.SemaphoreType.DMA,
        ],
    )
    def gather(slot_hbm, hs_hbm, ctab_hbm, pos_hbm, out_hbm,
               slot_v, hs_v, pairs_v, out_v, pos_v, sem_g, sem_p):
        wid = lax.axis_index("s") * _NC + lax.axis_index("c")
        base = wid * b_per_w
        t0 = lax.rem(base, T)

        pltpu.sync_copy(slot_hbm.at[pl.ds(base, b_per_w)], slot_v)
        pltpu.sync_copy(hs_hbm.at[pl.ds(base, b_per_w)], hs_v)
        pos_cp = pltpu.async_copy(pos_hbm.at[pl.ds(t0, b_per_w)], pos_v, sem_p)
        gathers = []
        for k in range(b_per_w // 128):
            gathers.append(pltpu.async_copy(
                ctab_hbm.at[slot_v.at[pl.ds(k * 128, 128)]],
                pairs_v.at[pl.ds(k * 128, 128)],
                sem_g,
            ))
        pos_cp.wait()
        for g in gathers:
            g.wait()

        def row_block(ci, carry):
            row0 = ci * _L
            hv = hs_v[pl.ds(row0, _L)]
            for l in range(_L):
                st = hv[l]
                i = row0 + l
                for j in range(D // _L):
                    out_v[i, pl.ds(j * _L, _L)] = (
                        pairs_v[i, pl.ds(st + j * _L, _L)]
                        + pos_v[i, pl.ds(j * _L, _L)])
            return carry
        lax.fori_loop(0, b_per_w // _L, row_block, 0)

        pltpu.sync_copy(out_v, out_hbm.at[pl.ds(base, b_per_w)])

    ctab = tok_table.reshape(V // 2, 2 * D)
    slot = idx_flat >> 1
    hs = (idx_flat & 1) * jnp.int32(D)
    return gather(slot, hs, ctab, pos_table)


def kernel(idx, tok_table, pos_table):
    B, T = idx.shape
    V, D = tok_table.shape
    idx_flat = idx.reshape(-1).astype(jnp.int32)
    out = _embed_lookup(idx_flat, tok_table, pos_table, B, T, D)
    return out.reshape(B, T, D)


# speedup vs baseline: 1.7483x; 1.5975x over previous
import functools

import jax
import jax.numpy as jnp
from jax import lax
from jax.experimental import pallas as pl
from jax.experimental.pallas import tpu as pltpu
from jax.experimental.pallas import tpu_sc as plsc

_NC = 2
_NS = 16
_NW = _NC * _NS
_L = 16
_SEG = 32


@functools.partial(jax.jit, static_argnums=(3, 4, 5))
def _embed_lookup(idx_flat, tok_table, pos_table, B, T, D):
    n_tok = B * T
    V = tok_table.shape[0]
    b_per_w = n_tok // _NW
    n_seg = b_per_w // _SEG
    mesh = plsc.VectorSubcoreMesh(core_axis_name="c", subcore_axis_name="s")

    @functools.partial(
        pl.kernel,
        out_type=jax.ShapeDtypeStruct((n_tok, D), jnp.float32),
        mesh=mesh,
        scratch_types=[
            pltpu.VMEM((b_per_w,), jnp.int32),
            pltpu.VMEM((_SEG, 8, D), jnp.float32),
            pltpu.VMEM((b_per_w, D), jnp.float32),
            pltpu.VMEM((b_per_w, D), jnp.float32),
            pltpu.SemaphoreType.DMA,
            pltpu.SemaphoreType.DMA,
        ],
    )
    def body(idx_hbm, tok_hbm, pos_hbm, out_hbm,
             idx_v, groups_v, out_v, pos_v, sem_g, sem_p):
        wid = lax.axis_index("s") * _NC + lax.axis_index("c")
        base = wid * b_per_w
        t0 = lax.rem(base, T)
        tok3 = tok_hbm.reshape(V // 8, 8, D)

        pltpu.sync_copy(idx_hbm.at[pl.ds(base, b_per_w)], idx_v)
        pos_cp = pltpu.async_copy(pos_hbm.at[pl.ds(t0, b_per_w)], pos_v, sem_p)
        pos_cp.wait()

        for sgi in range(n_seg):
            copies = []
            for ci in range(_SEG // _L):
                v = lax.shift_right_logical(
                    idx_v[pl.ds(sgi * _SEG + ci * _L, _L)], 3)
                for l in range(_L):
                    copies.append(pltpu.async_copy(
                        tok3.at[pl.ds(v[l], 1)],
                        groups_v.at[pl.ds(ci * _L + l, 1)],
                        sem_g,
                    ))
            for cp in copies:
                cp.wait()

            def seg_body(ci, carry, sgi=sgi):
                row0 = sgi * _SEG + ci * _L
                sub = idx_v[pl.ds(row0, _L)] & 7
                for l in range(_L):
                    r = sub[l]
                    i = row0 + l
                    for j in range(D // _L):
                        s = pl.ds(j * _L, _L)
                        out_v[i, s] = groups_v[ci * _L + l, r, s] + pos_v[i, s]
                return carry
            lax.fori_loop(0, _SEG // _L, seg_body, 0)

        pltpu.sync_copy(out_v, out_hbm.at[pl.ds(base, b_per_w)])

    return body(idx_flat, tok_table, pos_table)


def kernel(idx, tok_table, pos_table):
    B, T = idx.shape
    V, D = tok_table.shape
    idx_flat = idx.reshape(-1).astype(jnp.int32)
    out = _embed_lookup(idx_flat, tok_table, pos_table, B, T, D)
    return out.reshape(B, T, D)
